# Initial kernel scaffold; baseline (speedup 1.0000x reference)
import jax
import jax.numpy as jnp
from jax.experimental import pallas as pl


def kernel(edge_lists, node_states, pos_lists, W, b):
    raise NotImplementedError("write your pallas kernel here")


# speedup vs baseline: 3.8855x; 3.8855x over previous
import functools

import jax
import jax.numpy as jnp
from jax import lax
from jax.experimental import pallas as pl
from jax.experimental.pallas import tpu as pltpu
from jax.experimental.pallas import tpu_sc as plsc

_SMALL = 1e-08
_NC = 2
_NS = 16
_LANES = 16


def _build_table(node_states, Wt, b2, T, N, D, DP, BN):
    def body(x_ref, wt_ref, b_ref, out_ref):
        mm = jnp.dot(x_ref[...], wt_ref[...], preferred_element_type=jnp.float32)
        out_ref[:, :D] = mm + b_ref[0]
        lane = lax.broadcasted_iota(jnp.int32, (BN, DP - D), 1)
        out_ref[:, D:] = jnp.where(lane == 0, 1.0, 0.0).astype(jnp.float32)

    return pl.pallas_call(
        body,
        grid=(T, N // BN),
        in_specs=[
            pl.BlockSpec((BN, D), lambda t, i: (i, 0)),
            pl.BlockSpec((D, D), lambda t, i: (0, t)),
            pl.BlockSpec((1, 1, D), lambda t, i: (t, 0, 0)),
        ],
        out_specs=pl.BlockSpec((BN, DP), lambda t, i: (t * (N // BN) + i, 0)),
        out_shape=jax.ShapeDtypeStruct((T * N, DP), jnp.float32),
    )(node_states, Wt, b2)


def _edge_scatter(table, src, dst, N, DP, E):
    EPW = E // (_NC * _NS)
    CH = 80
    RPT = N // _NS
    ZR = 125

    mesh = plsc.VectorSubcoreMesh(core_axis_name="c", subcore_axis_name="s")

    @functools.partial(
        pl.kernel,
        out_type=jax.ShapeDtypeStruct((_NC, N, DP), jnp.float32),
        mesh=mesh,
        scratch_types=[
            pltpu.VMEM_SHARED((N, DP), jnp.float32),
            pltpu.VMEM((CH,), jnp.int32),
            pltpu.VMEM((CH,), jnp.int32),
            pltpu.VMEM((CH, DP), jnp.float32),
            pltpu.VMEM((ZR, DP), jnp.float32),
        ],
        compiler_params=pltpu.CompilerParams(use_tc_tiling_on_sc=False),
    )
    def run(table_hbm, src_hbm, dst_hbm, out_hbm, acc, sidx, didx, rows, zbuf):
        cid = lax.axis_index("c")
        sid = lax.axis_index("s")
        wid = cid * _NS + sid
        row0 = sid * RPT

        @pl.loop(0, ZR)
        def _(r):
            @pl.loop(0, DP, step=_LANES)
            def _(c):
                zbuf[r, pl.ds(c, _LANES)] = jnp.zeros((_LANES,), jnp.float32)

        @pl.loop(0, RPT, step=ZR)
        def _(r):
            pltpu.sync_copy(zbuf, acc.at[pl.ds(row0 + r, ZR)])

        plsc.subcore_barrier()

        e0 = wid * EPW

        @pl.loop(0, EPW, step=CH)
        def _(i):
            pltpu.sync_copy(src_hbm.at[pl.ds(e0 + i, CH)], sidx)
            pltpu.sync_copy(dst_hbm.at[pl.ds(e0 + i, CH)], didx)
            pltpu.sync_copy(table_hbm.at[sidx], rows)
            pltpu.sync_copy(rows, acc.at[didx], add=True)

        plsc.subcore_barrier()

        @pl.loop(0, RPT, step=ZR)
        def _(r):
            pltpu.sync_copy(acc.at[pl.ds(row0 + r, ZR)],
                            out_hbm.at[cid, pl.ds(row0 + r, ZR)])

    return run(table, src, dst)


def _normalize(partials, N, D, DP, BN):
    def body(p_ref, o_ref):
        s = p_ref[0] + p_ref[1]
        cnt = jnp.sum(s[:, D:], axis=1, keepdims=True)
        div = jnp.where(cnt == 0.0, 1.0, cnt) + _SMALL
        o_ref[...] = s[:, :D] / div

    return pl.pallas_call(
        body,
        grid=(N // BN,),
        in_specs=[pl.BlockSpec((_NC, BN, DP), lambda i: (0, i, 0))],
        out_specs=pl.BlockSpec((BN, D), lambda i: (i, 0)),
        out_shape=jax.ShapeDtypeStruct((N, D), jnp.float32),
    )(partials)


def kernel(edge_lists, node_states, pos_lists, W, b):
    del pos_lists
    N, D = node_states.shape
    T, M, _ = edge_lists.shape
    E = T * M
    DP = D + _LANES

    offs = (jnp.arange(T, dtype=jnp.int32) * N)[:, None]
    src = (edge_lists[:, :, 0] + offs).reshape(E)
    dst = edge_lists[:, :, 1].reshape(E)
    Wt = W.T
    b2 = b.reshape(T, 1, D)

    table = _build_table(node_states, Wt, b2, T, N, D, DP, BN=1000)
    partials = _edge_scatter(table, src, dst, N, DP, E)
    return _normalize(partials, N, D, DP, BN=1000)


# speedup vs baseline: 6.7981x; 1.7496x over previous
import functools

import jax
import jax.numpy as jnp
from jax import lax
from jax.experimental import pallas as pl
from jax.experimental.pallas import tpu as pltpu
from jax.experimental.pallas import tpu_sc as plsc

_SMALL = 1e-08
_NC = 2
_NS = 16
_LANES = 16


def _build_table(node_states, Wt, b2, T, N, D, DP, BN):
    def body(x_ref, wt_ref, b_ref, out_ref):
        mm = jnp.dot(x_ref[...], wt_ref[...], preferred_element_type=jnp.float32)
        out_ref[:, :D] = mm + b_ref[0]
        lane = lax.broadcasted_iota(jnp.int32, (BN, DP - D), 1)
        out_ref[:, D:] = jnp.where(lane == 0, 1.0, 0.0).astype(jnp.float32)

    return pl.pallas_call(
        body,
        grid=(T, N // BN),
        in_specs=[
            pl.BlockSpec((BN, D), lambda t, i: (i, 0)),
            pl.BlockSpec((D, D), lambda t, i: (0, t)),
            pl.BlockSpec((1, 1, D), lambda t, i: (t, 0, 0)),
        ],
        out_specs=pl.BlockSpec((BN, DP), lambda t, i: (t * (N // BN) + i, 0)),
        out_shape=jax.ShapeDtypeStruct((T * N, DP), jnp.float32),
    )(node_states, Wt, b2)


def _edge_scatter(table, src, dst, N, DP, E):
    NW = _NC * _NS
    EPW = E // NW
    CH = 80
    NCH = EPW // CH
    BCH = 25
    NB = NCH // BCH
    EPB = BCH * CH
    RPT = N // _NS

    dst3 = dst.reshape(NW, NCH, CH)
    mesh = plsc.VectorSubcoreMesh(core_axis_name="c", subcore_axis_name="s")

    @functools.partial(
        pl.kernel,
        out_type=jax.ShapeDtypeStruct((_NC, N, DP), jnp.float32),
        mesh=mesh,
        scratch_types=[
            pltpu.VMEM_SHARED((N, DP), jnp.float32),
            pltpu.VMEM((EPB,), jnp.int32),
            pltpu.VMEM((BCH, CH), jnp.int32),
            pltpu.VMEM((CH, DP), jnp.float32),
            pltpu.VMEM((CH, DP), jnp.float32),
            pltpu.SemaphoreType.DMA,
            pltpu.SemaphoreType.DMA,
        ],
        compiler_params=pltpu.CompilerParams(use_tc_tiling_on_sc=False),
    )
    def run(table_hbm, src_hbm, dst_hbm, out_hbm, acc, sidx, didx,
            rows0, rows1, sem0, sem1):
        cid = lax.axis_index("c")
        sid = lax.axis_index("s")
        wid = cid * _NS + sid
        row0 = sid * RPT
        rows = (rows0, rows1)
        sems = (sem0, sem1)

        for b in range(2):
            @pl.loop(0, CH)
            def _(r):
                @pl.loop(0, DP, step=_LANES)
                def _(c):
                    rows[b][r, pl.ds(c, _LANES)] = jnp.zeros((_LANES,),
                                                             jnp.float32)

        @pl.loop(0, RPT - 65, step=CH)
        def _(r):
            pltpu.sync_copy(rows0, acc.at[pl.ds(row0 + r, CH)])
        pltpu.sync_copy(rows1.at[pl.ds(0, 65)],
                        acc.at[pl.ds(row0 + RPT - 65, 65)])
        plsc.subcore_barrier()

        def start_gather(j, b):
            pltpu.async_copy(table_hbm.at[sidx.at[pl.ds(j * CH, CH)]],
                             rows[b], sems[b])

        def wait_gather(b):
            pltpu.make_async_copy(table_hbm.at[sidx.at[pl.ds(0, CH)]],
                                  rows[b], sems[b]).wait()

        def scatter(j, b):
            pltpu.sync_copy(rows[b], acc.at[didx.at[j]], add=True)

        @pl.loop(0, NB)
        def _(blk):
            pltpu.sync_copy(src_hbm.at[pl.ds(wid * EPW + blk * EPB, EPB)],
                            sidx)
            pltpu.sync_copy(dst_hbm.at[wid, pl.ds(blk * BCH, BCH)], didx)

            start_gather(0, 0)

            @pl.loop(0, (BCH - 1) // 2)
            def _(i):
                j = 2 * i
                start_gather(j + 1, 1)
                wait_gather(0)
                scatter(j, 0)
                start_gather(j + 2, 0)
                wait_gather(1)
                scatter(j + 1, 1)

            wait_gather(0)
            scatter(BCH - 1, 0)

        plsc.subcore_barrier()

        pltpu.sync_copy(acc.at[pl.ds(row0, RPT)],
                        out_hbm.at[cid, pl.ds(row0, RPT)])

    return run(table, src, dst3)


def _normalize(partials, N, D, DP, BN):
    def body(p_ref, o_ref):
        s = p_ref[0] + p_ref[1]
        cnt = jnp.sum(s[:, D:], axis=1, keepdims=True)
        div = jnp.where(cnt == 0.0, 1.0, cnt) + _SMALL
        o_ref[...] = s[:, :D] / div

    return pl.pallas_call(
        body,
        grid=(N // BN,),
        in_specs=[pl.BlockSpec((_NC, BN, DP), lambda i: (0, i, 0))],
        out_specs=pl.BlockSpec((BN, D), lambda i: (i, 0)),
        out_shape=jax.ShapeDtypeStruct((N, D), jnp.float32),
    )(partials)


def kernel(edge_lists, node_states, pos_lists, W, b):
    del pos_lists
    N, D = node_states.shape
    T, M, _ = edge_lists.shape
    E = T * M
    DP = D + _LANES

    offs = (jnp.arange(T, dtype=jnp.int32) * N)[:, None]
    src = (edge_lists[:, :, 0] + offs).reshape(E)
    dst = edge_lists[:, :, 1].reshape(E)
    Wt = W.T
    b2 = b.reshape(T, 1, D)

    table = _build_table(node_states, Wt, b2, T, N, D, DP, BN=1000)
    partials = _edge_scatter(table, src, dst, N, DP, E)
    return _normalize(partials, N, D, DP, BN=1000)


# speedup vs baseline: 8.2014x; 1.2064x over previous
import functools

import jax
import jax.numpy as jnp
from jax import lax
from jax.experimental import pallas as pl
from jax.experimental.pallas import tpu as pltpu
from jax.experimental.pallas import tpu_sc as plsc

_SMALL = 1e-08
_NC = 2
_NS = 16
_LANES = 16


def _build_table(node_states, Wt, b2, T, N, D, BN):
    def body(x_ref, wt_ref, b_ref, out_ref):
        mm = jnp.dot(x_ref[...], wt_ref[...], preferred_element_type=jnp.float32)
        out_ref[...] = mm + b_ref[0]

    return pl.pallas_call(
        body,
        grid=(T, N // BN),
        in_specs=[
            pl.BlockSpec((BN, D), lambda t, i: (i, 0)),
            pl.BlockSpec((D, D), lambda t, i: (0, t)),
            pl.BlockSpec((1, 1, D), lambda t, i: (t, 0, 0)),
        ],
        out_specs=pl.BlockSpec((BN, D), lambda t, i: (t * (N // BN) + i, 0)),
        out_shape=jax.ShapeDtypeStruct((T * N, D), jnp.float32),
    )(node_states, Wt, b2)


def _edge_scatter(table, src, dst, N, D, E):
    NW = _NC * _NS
    EPW = E // NW
    CH = 80
    NCH = EPW // CH
    BCH = 25
    NB = NCH // BCH
    EPB = BCH * CH
    RPT = N // _NS
    CW = 16

    dst3 = dst.reshape(NW, NCH, CH)
    mesh = plsc.VectorSubcoreMesh(core_axis_name="c", subcore_axis_name="s")

    @functools.partial(
        pl.kernel,
        out_type=(jax.ShapeDtypeStruct((_NC, N, D), jnp.float32),
                  jax.ShapeDtypeStruct((_NC, N, CW), jnp.float32)),
        mesh=mesh,
        scratch_types=[
            pltpu.VMEM_SHARED((N, D), jnp.float32),
            pltpu.VMEM_SHARED((N, CW), jnp.float32),
            pltpu.VMEM((EPB,), jnp.int32),
            pltpu.VMEM((BCH, CH), jnp.int32),
            pltpu.VMEM((CH, D), jnp.float32),
            pltpu.VMEM((CH, D), jnp.float32),
            pltpu.VMEM((CH, CW), jnp.float32),
            pltpu.SemaphoreType.DMA,
            pltpu.SemaphoreType.DMA,
            pltpu.SemaphoreType.DMA,
        ],
        compiler_params=pltpu.CompilerParams(use_tc_tiling_on_sc=False),
    )
    def run(table_hbm, src_hbm, dst_hbm, out_hbm, cnt_hbm, acc, cacc,
            sidx, didx, rows0, rows1, ones, sem0, sem1, semc):
        cid = lax.axis_index("c")
        sid = lax.axis_index("s")
        wid = cid * _NS + sid
        row0 = sid * RPT
        rows = (rows0, rows1)
        sems = (sem0, sem1)

        for b in range(2):
            @pl.loop(0, CH)
            def _(r):
                @pl.loop(0, D, step=_LANES)
                def _(c):
                    rows[b][r, pl.ds(c, _LANES)] = jnp.zeros((_LANES,),
                                                             jnp.float32)

        @pl.loop(0, CH)
        def _(r):
            ones[r, pl.ds(0, CW)] = jnp.zeros((CW,), jnp.float32)

        @pl.loop(0, RPT - 65, step=CH)
        def _(r):
            pltpu.sync_copy(rows0, acc.at[pl.ds(row0 + r, CH)])
        pltpu.sync_copy(rows1.at[pl.ds(0, 65)],
                        acc.at[pl.ds(row0 + RPT - 65, 65)])

        @pl.loop(0, RPT - 65, step=CH)
        def _(r):
            pltpu.sync_copy(ones, cacc.at[pl.ds(row0 + r, CH)])
        pltpu.sync_copy(ones.at[pl.ds(0, 65)],
                        cacc.at[pl.ds(row0 + RPT - 65, 65)])

        @pl.loop(0, CH)
        def _(r):
            ones[r, pl.ds(0, CW)] = jnp.ones((CW,), jnp.float32)

        plsc.subcore_barrier()

        def start_gather(j, b):
            pltpu.async_copy(table_hbm.at[sidx.at[pl.ds(j * CH, CH)]],
                             rows[b], sems[b])

        def wait_gather(b):
            pltpu.make_async_copy(table_hbm.at[sidx.at[pl.ds(0, CH)]],
                                  rows[b], sems[b]).wait()

        def scatter(j, b):
            pltpu.async_copy(ones, cacc.at[didx.at[j]], semc, add=True)
            pltpu.sync_copy(rows[b], acc.at[didx.at[j]], add=True)
            pltpu.make_async_copy(ones, cacc.at[didx.at[j]], semc).wait()

        @pl.loop(0, NB)
        def _(blk):
            pltpu.sync_copy(src_hbm.at[pl.ds(wid * EPW + blk * EPB, EPB)],
                            sidx)
            pltpu.sync_copy(dst_hbm.at[wid, pl.ds(blk * BCH, BCH)], didx)

            start_gather(0, 0)

            @pl.loop(0, (BCH - 1) // 2)
            def _(i):
                j = 2 * i
                start_gather(j + 1, 1)
                wait_gather(0)
                scatter(j, 0)
                start_gather(j + 2, 0)
                wait_gather(1)
                scatter(j + 1, 1)

            wait_gather(0)
            scatter(BCH - 1, 0)

        plsc.subcore_barrier()

        pltpu.sync_copy(acc.at[pl.ds(row0, RPT)],
                        out_hbm.at[cid, pl.ds(row0, RPT)])
        pltpu.sync_copy(cacc.at[pl.ds(row0, RPT)],
                        cnt_hbm.at[cid, pl.ds(row0, RPT)])

    return run(table, src, dst3)


def _normalize(partials, counts, N, D, CW, BN):
    def body(p_ref, c_ref, o_ref):
        s = p_ref[0] + p_ref[1]
        cnt = (c_ref[0] + c_ref[1])[:, :1]
        div = jnp.where(cnt == 0.0, 1.0, cnt) + _SMALL
        o_ref[...] = s / div

    return pl.pallas_call(
        body,
        grid=(N // BN,),
        in_specs=[
            pl.BlockSpec((_NC, BN, D), lambda i: (0, i, 0)),
            pl.BlockSpec((_NC, BN, CW), lambda i: (0, i, 0)),
        ],
        out_specs=pl.BlockSpec((BN, D), lambda i: (i, 0)),
        out_shape=jax.ShapeDtypeStruct((N, D), jnp.float32),
    )(partials, counts)


def kernel(edge_lists, node_states, pos_lists, W, b):
    del pos_lists
    N, D = node_states.shape
    T, M, _ = edge_lists.shape
    E = T * M

    offs = (jnp.arange(T, dtype=jnp.int32) * N)[:, None]
    src = (edge_lists[:, :, 0] + offs).reshape(E)
    dst = edge_lists[:, :, 1].reshape(E)
    Wt = W.T
    b2 = b.reshape(T, 1, D)

    table = _build_table(node_states, Wt, b2, T, N, D, BN=1000)
    partials, counts = _edge_scatter(table, src, dst, N, D, E)
    return _normalize(partials, counts, N, D, CW=16, BN=1000)
